# baseline (speedup 1.0000x reference)
import dataclasses
import functools

import jax
import jax.numpy as jnp
from jax import lax
from jax.experimental import pallas as pl
from jax.experimental.pallas import tpu as pltpu
from jax.experimental.pallas import tpu_sc as plsc

B = 4096 * 200
D = 64
LANES = 16
NC, NS = 2, 16
NW = NC * NS
BPW = B // NW
C = 512
K = C // 128
STEPS = BPW // C
IDX_ROWS = B // 128

_mesh = plsc.VectorSubcoreMesh(core_axis_name="c", subcore_axis_name="s")

_cp = pltpu.CompilerParams(needs_layout_passes=False, use_tc_tiling_on_sc=False)


@functools.partial(
    pl.kernel,
    compiler_params=_cp,
    out_type=jax.ShapeDtypeStruct((B, D), jnp.float32),
    mesh=_mesh,
    scratch_types=[
        pltpu.VMEM((K, 128), jnp.int32),
        pltpu.VMEM((K, 128), jnp.int32),
        pltpu.VMEM((C, D), jnp.float32),
        pltpu.VMEM((C, D), jnp.float32),
        pltpu.SemaphoreType.DMA,
        pltpu.SemaphoreType.DMA,
    ],
)
def _embed_lookup(table_hbm, idx_hbm, out_hbm,
                  idx0, idx1, rows0, rows1, sem0, sem1):
    wid = lax.axis_index("s") * NC + lax.axis_index("c")
    idx_row0 = wid * (BPW // 128)
    out_row0 = wid * BPW

    def load_and_fire(s, idx_v, rows_v, sem):
        pltpu.sync_copy(idx_hbm.at[pl.ds(idx_row0 + s * K, K)], idx_v)
        for j in range(K):
            pltpu.async_copy(
                table_hbm.at[idx_v.at[j]],
                rows_v.at[pl.ds(j * 128, 128)],
                sem,
            )

    def drain(rows_v, sem):
        pltpu.make_async_copy(table_hbm.at[pl.ds(0, C)], rows_v, sem).wait()

    def fixup(idx_v, rows_v):
        acc = idx_v[0, pl.ds(0, LANES)]
        for g in range(1, C // LANES):
            acc = jnp.minimum(acc, idx_v[g // 8, pl.ds((g % 8) * LANES, LANES)])

        @pl.when(jnp.min(acc) == 0)
        def _():
            zeros = jnp.zeros((LANES,), jnp.float32)
            for g in range(C // LANES):
                vec = idx_v[g // 8, pl.ds((g % 8) * LANES, LANES)]

                @pl.when(jnp.min(vec) == 0)
                def _():
                    mask = vec == 0
                    row_ids = lax.iota(jnp.int32, LANES) + (g * LANES)

                    @pl.loop(0, D)
                    def _(col):
                        col_ids = jnp.full((LANES,), 0, jnp.int32) + col
                        plsc.store_scatter(rows_v, [row_ids, col_ids],
                                           zeros, mask=mask)

    def retire(s, idx_v, rows_v, sem):
        drain(rows_v, sem)
        fixup(idx_v, rows_v)
        pltpu.sync_copy(rows_v, out_hbm.at[pl.ds(out_row0 + s * C, C)])

    load_and_fire(0, idx0, rows0, sem0)

    @pl.loop(0, STEPS // 2)
    def _(h):
        s0 = h * 2
        s1 = s0 + 1
        load_and_fire(s1, idx1, rows1, sem1)
        retire(s0, idx0, rows0, sem0)

        @pl.when(s1 + 1 < STEPS)
        def _():
            load_and_fire(s1 + 1, idx0, rows0, sem0)

        retire(s1, idx1, rows1, sem1)


def kernel(x, table):
    idx = x.astype(jnp.int32).reshape(IDX_ROWS, 128)
    out = _embed_lookup(table, idx)
    return out.reshape(x.shape + (D,))
